# baseline (speedup 1.0000x reference)
import functools

import jax
import jax.numpy as jnp
from jax import lax
from jax.experimental import pallas as pl
from jax.experimental.pallas import tpu as pltpu

_DN_RT = (((1,), (1,)), ((), ()))
_DN = (((1,), (0,)), ((), ()))


def _dist_min_kernel(yt_ref, x_ref, out_ref, tot_ref, acc_ref, bt_ref, *,
                     n_keys):
    j = pl.program_id(0)
    nk = pl.num_programs(0)
    kb = yt_ref.shape[1]

    @pl.when(j == 0)
    def _init():
        acc_ref[...] = jnp.full(acc_ref.shape, jnp.inf, acc_ref.dtype)
        x = x_ref[...]
        bt_ref[...] = jnp.concatenate(
            [(-2.0 * x).astype(jnp.bfloat16),
             jnp.ones(x.shape, jnp.bfloat16)], axis=1)

    yt = yt_ref[...]
    col = jax.lax.broadcasted_iota(jnp.int32, yt.shape, 1) + j * kb
    yt = jnp.where(col < n_keys, yt, 1e4).astype(jnp.bfloat16)
    a = jnp.concatenate([yt, yt * yt], axis=0)
    d = lax.dot_general(bt_ref[...], a, _DN,
                        preferred_element_type=jnp.float32)
    m = jnp.min(d.reshape(d.shape[0], -1, 128), axis=1)
    acc_ref[...] = jnp.minimum(acc_ref[...], m)

    @pl.when(j == nk - 1)
    def _finish():
        x = x_ref[...]
        x2 = lax.dot_general(jnp.ones((1, x.shape[1]), jnp.float32), x * x,
                             _DN_RT,
                             preferred_element_type=jnp.float32)
        accT = jnp.swapaxes(acc_ref[...], 0, 1)
        r = jnp.min(accT, axis=0, keepdims=True) + x2
        out_ref[...] = r
        tot_ref[...] = jnp.sum(r).reshape(1, 1)


@functools.partial(jax.jit, static_argnames=())
def kernel(x, y):
    q, dim = x.shape
    k = y.shape[0]
    yt = y.T
    kb = 5120
    nk = pl.cdiv(k, kb)

    closest_row, tot = pl.pallas_call(
        functools.partial(_dist_min_kernel, n_keys=k),
        grid=(nk,),
        in_specs=[
            pl.BlockSpec((dim, kb), lambda j: (0, j)),
            pl.BlockSpec((q, dim), lambda j: (0, 0)),
        ],
        out_specs=[
            pl.BlockSpec((1, q), lambda j: (0, 0)),
            pl.BlockSpec((1, 1), lambda j: (0, 0)),
        ],
        out_shape=[
            jax.ShapeDtypeStruct((1, q), jnp.float32),
            jax.ShapeDtypeStruct((1, 1), jnp.float32),
        ],
        scratch_shapes=[
            pltpu.VMEM((q, 128), jnp.float32),
            pltpu.VMEM((q, 2 * dim), jnp.bfloat16),
        ],
        compiler_params=pltpu.CompilerParams(
            dimension_semantics=("arbitrary",),
        ),
    )(yt, x)

    return (tot.reshape(()), closest_row.reshape(q))


# speedup vs baseline: 1.3635x; 1.3635x over previous
import functools

import jax
import jax.numpy as jnp
from jax import lax
from jax.experimental import pallas as pl
from jax.experimental.pallas import tpu as pltpu

_DN_RT = (((1,), (1,)), ((), ()))
_DN = (((1,), (0,)), ((), ()))
_CH = 512


def _dist_min_kernel(yt_ref, x_ref, out_ref, tot_ref, acc_ref, bt_ref, *,
                     n_keys):
    j = pl.program_id(0)
    nk = pl.num_programs(0)
    kb = yt_ref.shape[1]

    @pl.when(j == 0)
    def _init():
        acc_ref[...] = jnp.full(acc_ref.shape, jnp.inf, acc_ref.dtype)
        x = x_ref[...]
        bt_ref[...] = jnp.concatenate(
            [(-2.0 * x).astype(jnp.bfloat16),
             jnp.ones(x.shape, jnp.bfloat16)], axis=1)

    ytf = yt_ref[...]
    col = jax.lax.broadcasted_iota(jnp.int32, ytf.shape, 1)
    ytf = jnp.where(col < n_keys - j * kb, ytf, 1e4)
    bt = bt_ref[...]
    acc = acc_ref[...]
    for c in range(kb // _CH):
        ytT = jnp.swapaxes(ytf[:, c * _CH:(c + 1) * _CH], 0, 1)
        a = jnp.concatenate([ytT, ytT * ytT],
                            axis=1).astype(jnp.bfloat16)
        d = lax.dot_general(a, bt, _DN_RT,
                            preferred_element_type=jnp.float32)
        m = jnp.min(d.reshape(-1, 8, d.shape[1]), axis=0)
        acc = jnp.minimum(acc, m)
    acc_ref[...] = acc

    @pl.when(j == nk - 1)
    def _finish():
        x = x_ref[...]
        x2 = lax.dot_general(jnp.ones((1, x.shape[1]), jnp.float32), x * x,
                             _DN_RT,
                             preferred_element_type=jnp.float32)
        r = jnp.min(acc_ref[...], axis=0, keepdims=True) + x2
        out_ref[...] = r
        tot_ref[...] = jnp.sum(r).reshape(1, 1)


@functools.partial(jax.jit, static_argnames=())
def kernel(x, y):
    q, dim = x.shape
    k = y.shape[0]
    yt = y.T
    kb = 5120
    nk = pl.cdiv(k, kb)

    closest_row, tot = pl.pallas_call(
        functools.partial(_dist_min_kernel, n_keys=k),
        grid=(nk,),
        in_specs=[
            pl.BlockSpec((dim, kb), lambda j: (0, j)),
            pl.BlockSpec((q, dim), lambda j: (0, 0)),
        ],
        out_specs=[
            pl.BlockSpec((1, q), lambda j: (0, 0)),
            pl.BlockSpec((1, 1), lambda j: (0, 0)),
        ],
        out_shape=[
            jax.ShapeDtypeStruct((1, q), jnp.float32),
            jax.ShapeDtypeStruct((1, 1), jnp.float32),
        ],
        scratch_shapes=[
            pltpu.VMEM((8, q), jnp.float32),
            pltpu.VMEM((q, 2 * dim), jnp.bfloat16),
        ],
        compiler_params=pltpu.CompilerParams(
            dimension_semantics=("arbitrary",),
        ),
    )(yt, x)

    return (tot.reshape(()), closest_row.reshape(q))


# speedup vs baseline: 1.4001x; 1.0268x over previous
import functools

import jax
import jax.numpy as jnp
from jax import lax
from jax.experimental import pallas as pl
from jax.experimental.pallas import tpu as pltpu

_DN_RT = (((1,), (1,)), ((), ()))
_DN = (((1,), (0,)), ((), ()))
_CH = 512


def _dist_min_kernel(yt_ref, x_ref, out_ref, tot_ref, acc_ref, bt_ref, a_ref,
                     *, n_keys):
    j = pl.program_id(0)
    nk = pl.num_programs(0)
    kb = yt_ref.shape[1]

    @pl.when(j == 0)
    def _init():
        acc_ref[...] = jnp.full(acc_ref.shape, jnp.inf, acc_ref.dtype)
        x = x_ref[...]
        bt_ref[...] = jnp.concatenate(
            [(-2.0 * x).astype(jnp.bfloat16),
             jnp.ones(x.shape, jnp.bfloat16)], axis=1)

    ytf = yt_ref[...]
    col = jax.lax.broadcasted_iota(jnp.int32, ytf.shape, 1)
    ytf = jnp.where(col < n_keys - j * kb, ytf, 1e4)
    bt = bt_ref[...]
    acc = acc_ref[...]
    dim = ytf.shape[0]
    for c in range(kb // _CH):
        ytT = jnp.swapaxes(ytf[:, c * _CH:(c + 1) * _CH], 0, 1)
        lo, hi = c * _CH, (c + 1) * _CH
        a_ref[lo:hi, :dim] = ytT.astype(jnp.bfloat16)
        a_ref[lo:hi, dim:] = (ytT * ytT).astype(jnp.bfloat16)
        d = lax.dot_general(a_ref[lo:hi, :], bt, _DN_RT,
                            preferred_element_type=jnp.float32)
        m = jnp.min(d.reshape(-1, 8, d.shape[1]), axis=0)
        acc = jnp.minimum(acc, m)
    acc_ref[...] = acc

    @pl.when(j == nk - 1)
    def _finish():
        x = x_ref[...]
        x2 = lax.dot_general(jnp.ones((1, x.shape[1]), jnp.float32), x * x,
                             _DN_RT,
                             preferred_element_type=jnp.float32)
        r = jnp.min(acc_ref[...], axis=0, keepdims=True) + x2
        out_ref[...] = r
        tot_ref[...] = jnp.sum(r).reshape(1, 1)


@functools.partial(jax.jit, static_argnames=())
def kernel(x, y):
    q, dim = x.shape
    k = y.shape[0]
    yt = y.T
    kb = 5120
    nk = pl.cdiv(k, kb)

    closest_row, tot = pl.pallas_call(
        functools.partial(_dist_min_kernel, n_keys=k),
        grid=(nk,),
        in_specs=[
            pl.BlockSpec((dim, kb), lambda j: (0, j)),
            pl.BlockSpec((q, dim), lambda j: (0, 0)),
        ],
        out_specs=[
            pl.BlockSpec((1, q), lambda j: (0, 0)),
            pl.BlockSpec((1, 1), lambda j: (0, 0)),
        ],
        out_shape=[
            jax.ShapeDtypeStruct((1, q), jnp.float32),
            jax.ShapeDtypeStruct((1, 1), jnp.float32),
        ],
        scratch_shapes=[
            pltpu.VMEM((8, q), jnp.float32),
            pltpu.VMEM((q, 2 * dim), jnp.bfloat16),
            pltpu.VMEM((kb, 2 * dim), jnp.bfloat16),
        ],
        compiler_params=pltpu.CompilerParams(
            dimension_semantics=("arbitrary",),
        ),
    )(yt, x)

    return (tot.reshape(()), closest_row.reshape(q))


# speedup vs baseline: 1.4557x; 1.0397x over previous
import functools

import jax
import jax.numpy as jnp
from jax import lax
from jax.experimental import pallas as pl
from jax.experimental.pallas import tpu as pltpu

_DN_RT = (((1,), (1,)), ((), ()))
_DN = (((1,), (0,)), ((), ()))
_CH = 512


def _dist_min_kernel(yt_ref, x_ref, out_ref, tot_ref, acc_ref, bt_ref, a_ref,
                     *, n_keys):
    j = pl.program_id(0)
    nk = pl.num_programs(0)
    kb = yt_ref.shape[1]

    @pl.when(j == 0)
    def _init():
        acc_ref[...] = jnp.full(acc_ref.shape, jnp.inf, acc_ref.dtype)
        x = x_ref[...]
        bt_ref[...] = jnp.concatenate(
            [(-2.0 * x).astype(jnp.bfloat16),
             jnp.ones(x.shape, jnp.bfloat16)], axis=1)

    ytf = yt_ref[...]
    bt = bt_ref[...]
    acc = acc_ref[...]
    dim = ytf.shape[0]
    nkk = (n_keys + kb - 1) // kb
    for c in range(kb // _CH):
        lo, hi = c * _CH, (c + 1) * _CH
        yts = ytf[:, lo:hi]
        if (nkk - 1) * kb + hi > n_keys:
            col = jax.lax.broadcasted_iota(jnp.int32, yts.shape, 1)
            yts = jnp.where(col < n_keys - j * kb - lo, yts, 1e4)
        ytT = jnp.swapaxes(yts, 0, 1)
        a_ref[lo:hi, :dim] = ytT.astype(jnp.bfloat16)
        a_ref[lo:hi, dim:] = (ytT * ytT).astype(jnp.bfloat16)
        d = lax.dot_general(a_ref[lo:hi, :], bt, _DN_RT,
                            preferred_element_type=jnp.float32)
        m = jnp.min(d.reshape(-1, 8, d.shape[1]), axis=0)
        acc = jnp.minimum(acc, m)
    acc_ref[...] = acc

    @pl.when(j == nk - 1)
    def _finish():
        x = x_ref[...]
        x2 = lax.dot_general(jnp.ones((1, x.shape[1]), jnp.float32), x * x,
                             _DN_RT,
                             preferred_element_type=jnp.float32)
        r = jnp.min(acc_ref[...], axis=0, keepdims=True) + x2
        out_ref[...] = r
        tot_ref[...] = jnp.sum(r).reshape(1, 1)


@functools.partial(jax.jit, static_argnames=())
def kernel(x, y):
    q, dim = x.shape
    k = y.shape[0]
    yt = y.T
    kb = 10240
    nk = pl.cdiv(k, kb)

    closest_row, tot = pl.pallas_call(
        functools.partial(_dist_min_kernel, n_keys=k),
        grid=(nk,),
        in_specs=[
            pl.BlockSpec((dim, kb), lambda j: (0, j)),
            pl.BlockSpec((q, dim), lambda j: (0, 0)),
        ],
        out_specs=[
            pl.BlockSpec((1, q), lambda j: (0, 0)),
            pl.BlockSpec((1, 1), lambda j: (0, 0)),
        ],
        out_shape=[
            jax.ShapeDtypeStruct((1, q), jnp.float32),
            jax.ShapeDtypeStruct((1, 1), jnp.float32),
        ],
        scratch_shapes=[
            pltpu.VMEM((8, q), jnp.float32),
            pltpu.VMEM((q, 2 * dim), jnp.bfloat16),
            pltpu.VMEM((kb, 2 * dim), jnp.bfloat16),
        ],
        compiler_params=pltpu.CompilerParams(
            dimension_semantics=("arbitrary",),
        ),
    )(yt, x)

    return (tot.reshape(()), closest_row.reshape(q))
